# baseline (speedup 1.0000x reference)
import functools

import jax
import jax.numpy as jnp
from jax import lax
from jax.experimental import pallas as pl
from jax.experimental.pallas import tpu as pltpu
from jax.experimental.pallas import tpu_sc as plsc

N = 10000
E = 160000
NC = 2
NS = 16
NW = NC * NS
EPAD = 163840
SLAB = 640
NPAD = NS * SLAB
WAUG = 144

_f32 = jnp.float32


def _make_seg_kernel(width, gb, nbuf):
    mesh = plsc.VectorSubcoreMesh(core_axis_name="c", subcore_axis_name="s")
    rows_per_w = EPAD // gb // NW
    n_pub = SLAB // gb

    @functools.partial(
        pl.kernel,
        mesh=mesh,
        compiler_params=pltpu.CompilerParams(use_tc_tiling_on_sc=False),
        out_type=jax.ShapeDtypeStruct((NC, NPAD, width), _f32),
        scratch_types=[
            pltpu.VMEM((rows_per_w, gb), jnp.int32),
            pltpu.VMEM((rows_per_w, gb), jnp.int32),
            pltpu.VMEM((nbuf, gb, width), _f32),
            pltpu.VMEM_SHARED((NPAD, width), _f32),
            [pltpu.SemaphoreType.DMA] * nbuf,
        ],
    )
    def seg(table, src3d, dst3d, zeros, out, srcv, dstv, rows, accum, gsem):
        cid = lax.axis_index("c")
        sid = lax.axis_index("s")
        wid = sid * NC + cid

        pltpu.sync_copy(zeros, rows.at[0])
        for c in range(n_pub):
            pltpu.sync_copy(rows.at[0], accum.at[pl.ds(sid * SLAB + c * gb, gb)])
        plsc.subcore_barrier()

        pltpu.sync_copy(src3d.at[wid], srcv)
        pltpu.sync_copy(dst3d.at[wid], dstv)

        def wait_gather(b):
            pltpu.make_async_copy(table.at[srcv.at[b]], rows.at[b], gsem[b]).wait()

        for b in range(nbuf):
            pltpu.async_copy(table.at[srcv.at[b]], rows.at[b], gsem[b])

        n_steps = rows_per_w // nbuf

        def step(g, carry):
            for b in range(nbuf):
                wait_gather(b)
                pltpu.sync_copy(rows.at[b], accum.at[dstv.at[g * nbuf + b]],
                                add=True)

                @pl.when(g < n_steps - 1)
                def _():
                    pltpu.async_copy(table.at[srcv.at[(g + 1) * nbuf + b]],
                                     rows.at[b], gsem[b])
            return carry

        lax.fori_loop(0, n_steps, step, 0)
        plsc.subcore_barrier()

        for _ in range(2):
            for c in range(n_pub):
                pltpu.sync_copy(accum.at[pl.ds(sid * SLAB + c * gb, gb)], rows.at[0])
                pltpu.sync_copy(rows.at[0], out.at[cid, pl.ds(sid * SLAB + c * gb, gb)])
            plsc.subcore_barrier()

    return seg


_seg144 = _make_seg_kernel(WAUG, 64, 2)
_seg16 = _make_seg_kernel(16, 128, 8)


def _k1_body(s0_ref, x_ref, w0l_ref, w0r_ref, b0_ref, h0_ref, deg_ref):
    s0 = s0_ref[0] + s0_ref[1]
    deg = s0[:N, 128:129]
    mean0 = s0[:N, :128] / jnp.maximum(deg, 1.0)
    h0 = jnp.maximum(jnp.dot(mean0, w0l_ref[...])
                     + jnp.dot(x_ref[...], w0r_ref[...]) + b0_ref[...], 0.0)
    h0_ref[...] = h0
    deg_ref[...] = deg


def _k2_body(a1_ref, h0_ref, deg_ref, w1l_ref, w1r_ref, b1_ref, h1_ref):
    mean1 = (a1_ref[0][:N] + a1_ref[1][:N]) / jnp.maximum(deg_ref[...], 1.0)
    h1_ref[...] = jnp.maximum(jnp.dot(mean1, w1l_ref[...])
                              + jnp.dot(h0_ref[...], w1r_ref[...])
                              + b1_ref[...], 0.0)


def _k3_body(a2_ref, h1_ref, deg_ref, wel_ref, wer_ref, be_ref, wpl_ref,
             wpr_ref, bp_ref, w3l_ref, w3r_ref, b3_ref, w4l_ref, w4r_ref,
             b4_ref, wfl_ref, wfr_ref, bf_ref, wro_ref, bro_ref, out_ref):
    h1 = h1_ref[...]
    mean2 = (a2_ref[0][:N] + a2_ref[1][:N]) / jnp.maximum(deg_ref[...], 1.0)
    z = jnp.maximum(jnp.dot(mean2, wel_ref[...])
                    + jnp.dot(h1, wer_ref[...]) + be_ref[...], 0.0)
    sl = jnp.dot(mean2, wpl_ref[...]) + jnp.dot(h1, wpr_ref[...]) + bp_ref[...]
    m = jnp.max(sl, axis=1, keepdims=True)
    e = jnp.exp(sl - m)
    s = e / jnp.sum(e, axis=1, keepdims=True)

    hp = lax.dot_general(s, z, (((0,), (0,)), ((), ())))

    mh = jnp.sum(hp, axis=0, keepdims=True) / 10.0
    h3 = jnp.maximum(jnp.dot(mh, w3l_ref[...])
                     + jnp.dot(hp, w3r_ref[...]) + b3_ref[...], 0.0)
    mh3 = jnp.sum(h3, axis=0, keepdims=True) / 10.0
    h4 = jnp.maximum(jnp.dot(mh3, w4l_ref[...])
                     + jnp.dot(h3, w4r_ref[...]) + b4_ref[...], 0.0)
    mh4 = jnp.sum(h4, axis=0, keepdims=True) / 10.0
    fz = jnp.maximum(jnp.dot(mh4, wfl_ref[...])
                     + jnp.dot(h4, wfr_ref[...]) + bf_ref[...], 0.0)
    xfin = jnp.sum(fz, axis=0, keepdims=True)
    out_ref[...] = jnp.dot(xfin, wro_ref[...]) + bro_ref[...]


def _tc_call(body, out_shapes):
    return pl.pallas_call(body, out_shape=out_shapes)


def kernel(x, edge_index, W0l, W0r, b0, W1l, W1r, b1, Wel, Wer, be, Wpl, Wpr,
           bp, W3l, W3r, b3, W4l, W4r, b4, Wfl, Wfr, bf, Wql, Wqr, bq, Wro, bro):
    pad = EPAD - E
    srcp = jnp.concatenate([edge_index[0], jnp.zeros((pad,), jnp.int32)])
    dstp = jnp.concatenate([edge_index[1], jnp.full((pad,), N, jnp.int32)])
    src64 = srcp.reshape(NW, EPAD // 64 // NW, 64)
    dst64 = dstp.reshape(NW, EPAD // 64 // NW, 64)
    src128 = srcp.reshape(NW, EPAD // 128 // NW, 128)
    dst128 = dstp.reshape(NW, EPAD // 128 // NW, 128)
    xaug = jnp.concatenate(
        [x, jnp.ones((N, 1), _f32), jnp.zeros((N, WAUG - 129), _f32)], axis=1)
    z144 = jnp.zeros((64, WAUG), _f32)
    z16 = jnp.zeros((128, 16), _f32)

    s0 = _seg144(xaug, src64, dst64, z144)

    h0, deg = _tc_call(_k1_body, [
        jax.ShapeDtypeStruct((N, 16), _f32),
        jax.ShapeDtypeStruct((N, 1), _f32),
    ])(s0, x, W0l, W0r, b0.reshape(1, 16))

    a1 = _seg16(h0, src128, dst128, z16)

    (h1,) = _tc_call(_k2_body, [jax.ShapeDtypeStruct((N, 16), _f32)])(
        a1, h0, deg, W1l, W1r, b1.reshape(1, 16))

    a2 = _seg16(h1, src128, dst128, z16)

    (out,) = _tc_call(_k3_body, [jax.ShapeDtypeStruct((1, 6), _f32)])(
        a2, h1, deg,
        Wel, Wer, be.reshape(1, 10), Wpl, Wpr, bp.reshape(1, 10),
        W3l, W3r, b3.reshape(1, 16), W4l, W4r, b4.reshape(1, 16),
        Wfl, Wfr, bf.reshape(1, 16), Wro, bro.reshape(1, 6))
    return out


# speedup vs baseline: 1.0936x; 1.0936x over previous
import functools

import jax
import jax.numpy as jnp
from jax import lax
from jax.experimental import pallas as pl
from jax.experimental.pallas import tpu as pltpu
from jax.experimental.pallas import tpu_sc as plsc

N = 10000
E = 160000
NC = 2
NS = 16
NW = NC * NS
EPAD = 163840
SLAB = 640
NPAD = NS * SLAB
WAUG = 144

_f32 = jnp.float32


def _make_seg_kernel(width, gb, nbuf):
    mesh = plsc.VectorSubcoreMesh(core_axis_name="c", subcore_axis_name="s")
    rows_per_w = EPAD // gb // NW
    n_pub = SLAB // gb

    @functools.partial(
        pl.kernel,
        mesh=mesh,
        compiler_params=pltpu.CompilerParams(use_tc_tiling_on_sc=False),
        out_type=jax.ShapeDtypeStruct((NC, NPAD, width), _f32),
        scratch_types=[
            pltpu.VMEM((rows_per_w, gb), jnp.int32),
            pltpu.VMEM((rows_per_w, gb), jnp.int32),
            pltpu.VMEM((nbuf, gb, width), _f32),
            pltpu.VMEM_SHARED((NPAD, width), _f32),
            [pltpu.SemaphoreType.DMA] * nbuf,
        ],
    )
    def seg(table, src3d, dst3d, zeros, out, srcv, dstv, rows, accum, gsem):
        cid = lax.axis_index("c")
        sid = lax.axis_index("s")
        wid = sid * NC + cid

        pltpu.sync_copy(zeros, rows.at[0])
        for c in range(n_pub):
            pltpu.sync_copy(rows.at[0], accum.at[pl.ds(sid * SLAB + c * gb, gb)])
        plsc.subcore_barrier()

        pltpu.sync_copy(src3d.at[wid], srcv)
        pltpu.sync_copy(dst3d.at[wid], dstv)

        def wait_gather(b):
            pltpu.make_async_copy(table.at[srcv.at[b]], rows.at[b], gsem[b]).wait()

        for b in range(nbuf):
            pltpu.async_copy(table.at[srcv.at[b]], rows.at[b], gsem[b])

        n_steps = rows_per_w // nbuf

        def step(g, carry):
            for b in range(nbuf):
                wait_gather(b)
                pltpu.sync_copy(rows.at[b], accum.at[dstv.at[g * nbuf + b]],
                                add=True)

                @pl.when(g < n_steps - 1)
                def _():
                    pltpu.async_copy(table.at[srcv.at[(g + 1) * nbuf + b]],
                                     rows.at[b], gsem[b])
            return carry

        lax.fori_loop(0, n_steps, step, 0)
        plsc.subcore_barrier()

        for _ in range(2):
            for c in range(n_pub):
                pltpu.sync_copy(accum.at[pl.ds(sid * SLAB + c * gb, gb)], rows.at[0])
                pltpu.sync_copy(rows.at[0], out.at[cid, pl.ds(sid * SLAB + c * gb, gb)])
            plsc.subcore_barrier()

    return seg


_seg128 = _make_seg_kernel(128, 64, 4)
_seg16 = _make_seg_kernel(16, 128, 8)


def _make_deg_kernel():
    mesh = plsc.VectorSubcoreMesh(core_axis_name="c", subcore_axis_name="s")
    gb = 128
    rows_per_w = EPAD // gb // NW
    n_pub = SLAB // gb

    @functools.partial(
        pl.kernel,
        mesh=mesh,
        compiler_params=pltpu.CompilerParams(use_tc_tiling_on_sc=False),
        out_type=jax.ShapeDtypeStruct((NC, NPAD, 16), _f32),
        scratch_types=[
            pltpu.VMEM((rows_per_w, gb), jnp.int32),
            pltpu.VMEM((gb, 16), _f32),
            pltpu.VMEM_SHARED((NPAD, 16), _f32),
        ],
    )
    def deg(ones, dst3d, zeros, out, dstv, onev, accum):
        cid = lax.axis_index("c")
        sid = lax.axis_index("s")
        wid = sid * NC + cid

        pltpu.sync_copy(zeros, onev)
        for c in range(n_pub):
            pltpu.sync_copy(onev, accum.at[pl.ds(sid * SLAB + c * gb, gb)])
        plsc.subcore_barrier()

        pltpu.sync_copy(dst3d.at[wid], dstv)
        pltpu.sync_copy(ones, onev)

        def step(j, carry):
            pltpu.sync_copy(onev, accum.at[dstv.at[j]], add=True)
            return carry

        lax.fori_loop(0, rows_per_w, step, 0)
        plsc.subcore_barrier()

        for _ in range(2):
            for c in range(n_pub):
                pltpu.sync_copy(accum.at[pl.ds(sid * SLAB + c * gb, gb)], onev)
                pltpu.sync_copy(onev, out.at[cid, pl.ds(sid * SLAB + c * gb, gb)])
            plsc.subcore_barrier()

    return deg


_segdeg = _make_deg_kernel()


def _k1_body(s0_ref, dg_ref, x_ref, w0l_ref, w0r_ref, b0_ref, h0_ref, deg_ref):
    s0 = s0_ref[0] + s0_ref[1]
    deg = dg_ref[0][:N, :1] + dg_ref[1][:N, :1]
    mean0 = s0[:N] / jnp.maximum(deg, 1.0)
    h0 = jnp.maximum(jnp.dot(mean0, w0l_ref[...])
                     + jnp.dot(x_ref[...], w0r_ref[...]) + b0_ref[...], 0.0)
    h0_ref[...] = h0
    deg_ref[...] = deg


def _k2_body(a1_ref, h0_ref, deg_ref, w1l_ref, w1r_ref, b1_ref, h1_ref):
    mean1 = (a1_ref[0][:N] + a1_ref[1][:N]) / jnp.maximum(deg_ref[...], 1.0)
    h1_ref[...] = jnp.maximum(jnp.dot(mean1, w1l_ref[...])
                              + jnp.dot(h0_ref[...], w1r_ref[...])
                              + b1_ref[...], 0.0)


def _k3_body(a2_ref, h1_ref, deg_ref, wel_ref, wer_ref, be_ref, wpl_ref,
             wpr_ref, bp_ref, w3l_ref, w3r_ref, b3_ref, w4l_ref, w4r_ref,
             b4_ref, wfl_ref, wfr_ref, bf_ref, wro_ref, bro_ref, out_ref):
    h1 = h1_ref[...]
    mean2 = (a2_ref[0][:N] + a2_ref[1][:N]) / jnp.maximum(deg_ref[...], 1.0)
    z = jnp.maximum(jnp.dot(mean2, wel_ref[...])
                    + jnp.dot(h1, wer_ref[...]) + be_ref[...], 0.0)
    sl = jnp.dot(mean2, wpl_ref[...]) + jnp.dot(h1, wpr_ref[...]) + bp_ref[...]
    m = jnp.max(sl, axis=1, keepdims=True)
    e = jnp.exp(sl - m)
    s = e / jnp.sum(e, axis=1, keepdims=True)

    hp = lax.dot_general(s, z, (((0,), (0,)), ((), ())))

    mh = jnp.sum(hp, axis=0, keepdims=True) / 10.0
    h3 = jnp.maximum(jnp.dot(mh, w3l_ref[...])
                     + jnp.dot(hp, w3r_ref[...]) + b3_ref[...], 0.0)
    mh3 = jnp.sum(h3, axis=0, keepdims=True) / 10.0
    h4 = jnp.maximum(jnp.dot(mh3, w4l_ref[...])
                     + jnp.dot(h3, w4r_ref[...]) + b4_ref[...], 0.0)
    mh4 = jnp.sum(h4, axis=0, keepdims=True) / 10.0
    fz = jnp.maximum(jnp.dot(mh4, wfl_ref[...])
                     + jnp.dot(h4, wfr_ref[...]) + bf_ref[...], 0.0)
    xfin = jnp.sum(fz, axis=0, keepdims=True)
    out_ref[...] = jnp.dot(xfin, wro_ref[...]) + bro_ref[...]


def _tc_call(body, out_shapes):
    return pl.pallas_call(body, out_shape=out_shapes)


def kernel(x, edge_index, W0l, W0r, b0, W1l, W1r, b1, Wel, Wer, be, Wpl, Wpr,
           bp, W3l, W3r, b3, W4l, W4r, b4, Wfl, Wfr, bf, Wql, Wqr, bq, Wro, bro):
    pad = EPAD - E
    srcp = jnp.concatenate([edge_index[0], jnp.zeros((pad,), jnp.int32)])
    dstp = jnp.concatenate([edge_index[1], jnp.full((pad,), N, jnp.int32)])
    src64 = srcp.reshape(NW, EPAD // 64 // NW, 64)
    dst64 = dstp.reshape(NW, EPAD // 64 // NW, 64)
    src128 = srcp.reshape(NW, EPAD // 128 // NW, 128)
    dst128 = dstp.reshape(NW, EPAD // 128 // NW, 128)
    z128 = jnp.zeros((64, 128), _f32)
    z16 = jnp.zeros((128, 16), _f32)
    ones16 = jnp.ones((128, 16), _f32)

    dg = _segdeg(ones16, dst128, z16)
    s0 = _seg128(x, src64, dst64, z128)

    h0, deg = _tc_call(_k1_body, [
        jax.ShapeDtypeStruct((N, 16), _f32),
        jax.ShapeDtypeStruct((N, 1), _f32),
    ])(s0, dg, x, W0l, W0r, b0.reshape(1, 16))

    a1 = _seg16(h0, src128, dst128, z16)

    (h1,) = _tc_call(_k2_body, [jax.ShapeDtypeStruct((N, 16), _f32)])(
        a1, h0, deg, W1l, W1r, b1.reshape(1, 16))

    a2 = _seg16(h1, src128, dst128, z16)

    (out,) = _tc_call(_k3_body, [jax.ShapeDtypeStruct((1, 6), _f32)])(
        a2, h1, deg,
        Wel, Wer, be.reshape(1, 10), Wpl, Wpr, bp.reshape(1, 10),
        W3l, W3r, b3.reshape(1, 16), W4l, W4r, b4.reshape(1, 16),
        Wfl, Wfr, bf.reshape(1, 16), Wro, bro.reshape(1, 6))
    return out
